# baseline (speedup 1.0000x reference)
import functools

import jax
import jax.numpy as jnp
from jax import lax
from jax.experimental import pallas as pl
from jax.experimental.pallas import tpu as pltpu
from jax.experimental.pallas import tpu_sc as plsc

BATCH = 16384
FACTORS = 32
NUM_WORKERS = 32
B_PER_W = BATCH // NUM_WORKERS
LANES = 16

COLS_PER_W = 31232
CHUNK = 512
N_CHUNKS = COLS_PER_W // CHUNK
EXTRA_LO = 999424
TAIL_LO = 999936
USERS = 1000000
DUMP_ROW = BATCH
UG_ROWS = BATCH + 8


def _sweep_body(user_hbm, utT_hbm, utail_hbm, ug_hbm,
                ids_v, hid_v, hpos_v, mcol_v, mpos_v,
                cbufA, cbufB, tbuf, stg, pbuf,
                sem_i, sem_a, sem_b, sem_s):
    wid = lax.axis_index("s") * 2 + lax.axis_index("c")
    lo = wid * COLS_PER_W
    is_last = wid == NUM_WORKERS - 1
    hi = jnp.where(is_last, USERS, lo + COLS_PER_W)

    pltpu.async_copy(user_hbm.at[:], ids_v, sem_i).wait()

    it16 = lax.iota(jnp.int32, 16)

    def comp_body(k, nh):
        v = ids_v[pl.ds(k * 16, 16)]
        m = (v >= lo) & (v < hi)
        mi = m.astype(jnp.int32)
        ps = plsc.cumsum(mi)
        idx = jnp.maximum(nh + ps - 1, 0)
        plsc.store_scatter(hid_v, [idx], v, mask=m)
        plsc.store_scatter(hpos_v, [idx], it16 + k * 16, mask=m)
        return nh + jnp.sum(mi)

    nh = lax.fori_loop(0, BATCH // 16, comp_body, 0)
    pad_m = (nh + it16) < BATCH
    pad_i = jnp.minimum(nh + it16, BATCH - 1)
    plsc.store_scatter(hid_v, [pad_i], jnp.full((16,), 0x7FFFFFF, jnp.int32), mask=pad_m)
    nhv = (nh + 15) // 16

    def process(clo, buf, width):
        def mc_body(hv, mcnt):
            v = hid_v[pl.ds(hv * 16, 16)]
            m = (v >= clo) & (v < clo + width)
            mi = m.astype(jnp.int32)
            ps = plsc.cumsum(mi)
            idx = jnp.maximum(mcnt + ps - 1, 0)
            plsc.store_scatter(mcol_v, [idx], v - clo, mask=m)
            p = hpos_v[pl.ds(hv * 16, 16)]
            plsc.store_scatter(mpos_v, [idx], p, mask=m)
            return mcnt + jnp.sum(mi)

        mcnt = lax.fori_loop(0, nhv, mc_body, 0)
        pm = (mcnt + it16) < BATCH
        pi = jnp.minimum(mcnt + it16, BATCH - 1)
        plsc.store_scatter(mpos_v, [pi], jnp.full((16,), DUMP_ROW, jnp.int32), mask=pm)
        plsc.store_scatter(mcol_v, [pi], jnp.zeros((16,), jnp.int32), mask=pm)

        def g_body(g, carry):
            cols = mcol_v[pl.ds(g * 16, 16)]
            pos = mpos_v[pl.ds(g * 16, 16)]
            pbuf[...] = pos
            for f in range(FACTORS):
                fv = jnp.full((16,), f, jnp.int32)
                vals = plsc.load_gather(buf, [fv, cols])
                plsc.store_scatter(stg, [it16, fv], vals)
            pltpu.async_copy(stg, ug_hbm.at[pbuf], sem_s).wait()
            return carry

        lax.fori_loop(0, (mcnt + 15) // 16, g_body, 0)

    def issue(ct, buf, sem):
        c0 = pl.multiple_of(lo + ct * CHUNK, 128)
        pltpu.async_copy(utT_hbm.at[:, pl.ds(c0, CHUNK)], buf, sem)

    def drain(ct, buf, sem):
        c0 = pl.multiple_of(lo + ct * CHUNK, 128)
        pltpu.make_async_copy(utT_hbm.at[:, pl.ds(c0, CHUNK)], buf, sem).wait()

    issue(0, cbufA, sem_a)

    def sweep_body(g, carry):
        ct = g * 2
        issue(ct + 1, cbufB, sem_b)
        drain(ct, cbufA, sem_a)
        process(lo + ct * CHUNK, cbufA, CHUNK)
        issue(ct + 2, cbufA, sem_a)
        drain(ct + 1, cbufB, sem_b)
        process(lo + (ct + 1) * CHUNK, cbufB, CHUNK)
        return carry

    lax.fori_loop(0, (N_CHUNKS - 1) // 2, sweep_body, 0)
    drain(N_CHUNKS - 1, cbufA, sem_a)
    process(lo + (N_CHUNKS - 1) * CHUNK, cbufA, CHUNK)

    @pl.when(is_last)
    def _():
        pltpu.async_copy(utT_hbm.at[:, pl.ds(EXTRA_LO, CHUNK)], cbufB, sem_b)
        pltpu.make_async_copy(
            utT_hbm.at[:, pl.ds(EXTRA_LO, CHUNK)], cbufB, sem_b).wait()
        process(EXTRA_LO, cbufB, CHUNK)

        pltpu.async_copy(utail_hbm.at[:], tbuf, sem_b).wait()

        def mc_body(hv, mcnt):
            v = hid_v[pl.ds(hv * 16, 16)]
            m = (v >= TAIL_LO) & (v < USERS)
            mi = m.astype(jnp.int32)
            ps = plsc.cumsum(mi)
            idx = jnp.maximum(mcnt + ps - 1, 0)
            plsc.store_scatter(mcol_v, [idx], v - TAIL_LO, mask=m)
            p = hpos_v[pl.ds(hv * 16, 16)]
            plsc.store_scatter(mpos_v, [idx], p, mask=m)
            return mcnt + jnp.sum(mi)

        mcnt = lax.fori_loop(0, nhv, mc_body, 0)
        pm = (mcnt + it16) < BATCH
        pi = jnp.minimum(mcnt + it16, BATCH - 1)
        plsc.store_scatter(mpos_v, [pi], jnp.full((16,), DUMP_ROW, jnp.int32), mask=pm)
        plsc.store_scatter(mcol_v, [pi], jnp.zeros((16,), jnp.int32), mask=pm)

        def g_body(g, carry):
            d = mcol_v[pl.ds(g * 16, 16)]
            pos = mpos_v[pl.ds(g * 16, 16)]
            pbuf[...] = pos
            for f in range(FACTORS):
                w = d * FACTORS + f
                vals = plsc.load_gather(tbuf, [w >> 7, w & 127])
                plsc.store_scatter(stg, [it16, jnp.full((16,), f, jnp.int32)], vals)
            pltpu.async_copy(stg, ug_hbm.at[pbuf], sem_s).wait()
            return carry

        lax.fori_loop(0, (mcnt + 15) // 16, g_body, 0)


def _dot_body(item_hbm, itab_hbm, ug_hbm, out_hbm,
              iidx_v, ubuf, vrows, outv, sem_v, sem_u):
    wid = lax.axis_index("s") * 2 + lax.axis_index("c")
    base = wid * B_PER_W

    pltpu.sync_copy(item_hbm.at[pl.ds(base, B_PER_W)], iidx_v)
    cu = pltpu.async_copy(ug_hbm.at[pl.ds(base, B_PER_W), :], ubuf, sem_u)
    cv = pltpu.async_copy(itab_hbm.at[iidx_v], vrows, sem_v)
    cu.wait()
    cv.wait()

    lane = lax.iota(jnp.int32, 16)

    def group_body(g, carry):
        rows = lane + g * LANES
        acc = jnp.zeros((16,), jnp.float32)
        for f in range(FACTORS):
            cols = jnp.full((16,), f, jnp.int32)
            u = plsc.load_gather(ubuf, [rows, cols])
            v = plsc.load_gather(vrows, [rows, cols])
            acc = acc + u * v
        outv[pl.ds(pl.multiple_of(g * LANES, LANES), LANES)] = acc
        return carry

    lax.fori_loop(0, B_PER_W // LANES, group_body, 0)

    pltpu.sync_copy(outv, out_hbm.at[pl.ds(base, B_PER_W)])


def kernel(user, item, user_table, item_table):
    mesh = plsc.VectorSubcoreMesh(core_axis_name="c", subcore_axis_name="s")
    utail = user_table[TAIL_LO:].reshape(16, 128)

    sweep = functools.partial(
        pl.kernel,
        out_type=jax.ShapeDtypeStruct((UG_ROWS, 128), jnp.float32),
        mesh=mesh,
        compiler_params=pltpu.CompilerParams(needs_layout_passes=False),
        scratch_types=[
            pltpu.VMEM((BATCH,), jnp.int32),
            pltpu.VMEM((BATCH,), jnp.int32),
            pltpu.VMEM((BATCH,), jnp.int32),
            pltpu.VMEM((BATCH,), jnp.int32),
            pltpu.VMEM((BATCH,), jnp.int32),
            pltpu.VMEM((FACTORS, CHUNK), jnp.float32),
            pltpu.VMEM((FACTORS, CHUNK), jnp.float32),
            pltpu.VMEM((16, 128), jnp.float32),
            pltpu.VMEM((16, 128), jnp.float32),
            pltpu.VMEM((16,), jnp.int32),
            pltpu.SemaphoreType.DMA,
            pltpu.SemaphoreType.DMA,
            pltpu.SemaphoreType.DMA,
            pltpu.SemaphoreType.DMA,
        ],
    )(_sweep_body)
    ug = sweep(user, user_table.T, utail)

    dot = functools.partial(
        pl.kernel,
        out_type=jax.ShapeDtypeStruct((BATCH,), jnp.float32),
        mesh=mesh,
        compiler_params=pltpu.CompilerParams(
            needs_layout_passes=False, use_tc_tiling_on_sc=False),
        scratch_types=[
            pltpu.VMEM((B_PER_W,), jnp.int32),
            pltpu.VMEM((B_PER_W, 128), jnp.float32),
            pltpu.VMEM((B_PER_W, FACTORS), jnp.float32),
            pltpu.VMEM((B_PER_W,), jnp.float32),
            pltpu.SemaphoreType.DMA,
            pltpu.SemaphoreType.DMA,
        ],
    )(_dot_body)
    return dot(item, item_table, ug)
